# baseline (speedup 1.0000x reference)
import functools
import math

import jax
import jax.numpy as jnp
from jax import lax
from jax.experimental import pallas as pl
from jax.experimental.pallas import tpu as pltpu
from jax.experimental.pallas import tpu_sc as plsc

N = 10000
E = 320000
D = 128
DE = 16
C = 128

NC, NS, L = 2, 16, 16
NW = NC * NS
EPW = E // NW
CH = 40
NCHUNK = EPW // CH
ROW = 160
NP = 10240
RPT = NP // NS
ZCH = 40
OCH = 16
DKV = 2 * D
DQ2 = D + 2 * DE

BN = 2000


def _proj_body(x_ref, wq, bq, wk, bk, wvp, bvp, wsk, bsk, wet,
               q2_o, kv_o, sk_o):
    xb = x_ref[...]
    q = jnp.dot(xb, wq[...], preferred_element_type=jnp.float32) + bq[...]
    q2_o[:, :D] = q.astype(jnp.bfloat16)
    q2_o[:, D:] = jnp.dot(q, wet[...],
                          preferred_element_type=jnp.float32).astype(jnp.bfloat16)
    kv_o[:, :D] = (jnp.dot(xb, wk[...], preferred_element_type=jnp.float32)
                   + bk[...]).astype(jnp.bfloat16)
    kv_o[:, D:] = (jnp.dot(xb, wvp[...], preferred_element_type=jnp.float32)
                   + bvp[...]).astype(jnp.bfloat16)
    sk_o[...] = jnp.dot(xb, wsk[...], preferred_element_type=jnp.float32) + bsk[...]


_proj = pl.pallas_call(
    _proj_body,
    grid=(N // BN,),
    in_specs=[
        pl.BlockSpec((BN, D), lambda i: (i, 0)),
        pl.BlockSpec((D, C), lambda i: (0, 0)),
        pl.BlockSpec((1, C), lambda i: (0, 0)),
        pl.BlockSpec((D, C), lambda i: (0, 0)),
        pl.BlockSpec((1, C), lambda i: (0, 0)),
        pl.BlockSpec((D, C), lambda i: (0, 0)),
        pl.BlockSpec((1, C), lambda i: (0, 0)),
        pl.BlockSpec((D, C), lambda i: (0, 0)),
        pl.BlockSpec((1, C), lambda i: (0, 0)),
        pl.BlockSpec((D, 2 * DE), lambda i: (0, 0)),
    ],
    out_specs=[
        pl.BlockSpec((BN, DQ2), lambda i: (i, 0)),
        pl.BlockSpec((BN, DKV), lambda i: (i, 0)),
        pl.BlockSpec((BN, C), lambda i: (i, 0)),
    ],
    out_shape=[
        jax.ShapeDtypeStruct((N, DQ2), jnp.bfloat16),
        jax.ShapeDtypeStruct((N, DKV), jnp.bfloat16),
        jax.ShapeDtypeStruct((N, C), jnp.float32),
    ],
)


_mesh = plsc.VectorSubcoreMesh(core_axis_name="c", subcore_axis_name="s",
                               num_cores=NC, num_subcores=NS)

_F = plsc.PackFormat.INTERLEAVED


@functools.partial(
    pl.kernel,
    out_type=jax.ShapeDtypeStruct((NC, NP, ROW), jnp.float32),
    mesh=_mesh,
    compiler_params=pltpu.CompilerParams(needs_layout_passes=False,
                                         use_tc_tiling_on_sc=False),
    scratch_types=[
        pltpu.VMEM((2, CH), jnp.int32),
        pltpu.VMEM((2, CH), jnp.int32),
        pltpu.VMEM((CH,), jnp.int32),
        pltpu.VMEM((2, CH, DQ2), jnp.bfloat16),
        pltpu.VMEM((2, CH, DKV), jnp.bfloat16),
        pltpu.VMEM((2, CH, DE), jnp.float32),
        pltpu.VMEM((CH, ROW), jnp.float32),
        pltpu.VMEM_SHARED((NP, ROW), jnp.float32),
    ] + [pltpu.SemaphoreType.DMA] * 12,
)
def _edge_kernel(q2_hbm, kv_hbm, ea_hbm, ei_hbm,
                 zer_hbm, part_hbm, src_v, dst_v, dst_s, q2buf, kvbuf,
                 eabuf, msgbuf, acc, sems, semd,
                 semsrc0, semsrc1, semdst0, semdst1,
                 semkv0, semkv1, semq20, semq21, semea0, semea1):
    cid = lax.axis_index("c")
    sid = lax.axis_index("s")
    wid = cid * NS + sid
    row0 = sid * RPT

    inv_sqrt_c = jnp.float32(1.0 / math.sqrt(C))
    base_w = wid * EPW
    lane0 = lax.iota(jnp.int32, L) == 0
    lanes = lax.iota(jnp.int32, L)
    shuf = [(lanes ^ o)[:, None] for o in (8, 4, 2, 1)]
    gdn = lax.GatherDimensionNumbers(offset_dims=(), collapsed_slice_dims=(0,),
                                     start_index_map=(0,))

    def lane_perm(x, idx):
        return lax.gather(x, idx, gdn, (1,),
                          mode=lax.GatherScatterMode.PROMISE_IN_BOUNDS)

    semsrc = (semsrc0, semsrc1)
    semdst = (semdst0, semdst1)
    semkv = (semkv0, semkv1)
    semq2 = (semq20, semq21)
    semea = (semea0, semea1)


    def issue_idx(s, par):
        b0 = pl.multiple_of(base_w + s * CH, 8)
        pltpu.async_copy(ei_hbm.at[0, pl.ds(b0, CH)], src_v.at[par],
                         semsrc[par])
        pltpu.async_copy(ei_hbm.at[1, pl.ds(b0, CH)], dst_v.at[par],
                         semdst[par])

    def issue_gath(s, par):
        b0 = pl.multiple_of(base_w + s * CH, 8)
        pltpu.async_copy(ea_hbm.at[pl.ds(b0, CH)], eabuf.at[par], semea[par])
        pltpu.make_async_copy(ei_hbm.at[0, pl.ds(b0, CH)], src_v.at[par],
                              semsrc[par]).wait()
        pltpu.make_async_copy(ei_hbm.at[1, pl.ds(b0, CH)], dst_v.at[par],
                              semdst[par]).wait()
        pltpu.async_copy(kv_hbm.at[src_v.at[par]], kvbuf.at[par], semkv[par])
        pltpu.async_copy(q2_hbm.at[dst_v.at[par]], q2buf.at[par], semq2[par])

    def finish_chunk(s, par):
        pltpu.make_async_copy(kv_hbm.at[src_v.at[par]], kvbuf.at[par],
                              semkv[par]).wait()
        pltpu.make_async_copy(q2_hbm.at[dst_v.at[par]], q2buf.at[par],
                              semq2[par]).wait()
        pltpu.make_async_copy(ea_hbm.at[pl.ds(0, CH)], eabuf.at[par],
                              semea[par]).wait()
        @pl.when(s + 2 < NCHUNK)
        def _():
            issue_idx(s + 2, par)
        @pl.when(s > 0)
        def _():
            pltpu.make_async_copy(msgbuf, acc.at[dst_s], sems).wait()
        b0 = pl.multiple_of(base_w + s * CH, 8)
        cps = pltpu.async_copy(ei_hbm.at[1, pl.ds(b0, CH)], dst_s, semd)

        q2c = q2buf.at[par]
        kvc = kvbuf.at[par]
        eac = eabuf.at[par]

        @plsc.parallel_loop(0, CH, unroll=16)
        def edge_body(e):
            ea_e = eac[e, :]
            qe_a, _ = plsc.unpack(q2c[e, pl.ds(D, 2 * L)], format=_F,
                                  preferred_element_type=jnp.float32)
            part = qe_a * ea_e
            for cc in range(D // (2 * L)):
                qa, qb = plsc.unpack(q2c[e, pl.ds(2 * L * cc, 2 * L)],
                                     format=_F,
                                     preferred_element_type=jnp.float32)
                ka, kb = plsc.unpack(kvc[e, pl.ds(2 * L * cc, 2 * L)],
                                     format=_F,
                                     preferred_element_type=jnp.float32)
                part = part + qa * ka + qb * kb
            for sh in shuf:
                part = part + lane_perm(part, sh)
            ex = jnp.exp(part * inv_sqrt_c)
            for cc in range(D // (2 * L)):
                va, vb = plsc.unpack(kvc[e, pl.ds(D + 2 * L * cc, 2 * L)],
                                     format=_F,
                                     preferred_element_type=jnp.float32)
                msgbuf[e, pl.ds(2 * L * cc, L)] = va * ex
                msgbuf[e, pl.ds(2 * L * cc + L, L)] = vb * ex
            msgbuf[e, pl.ds(D, L)] = ea_e * ex
            msgbuf[e, pl.ds(D + DE, L)] = jnp.where(
                lane0, ex, jnp.zeros((L,), jnp.float32))

        cps.wait()
        pltpu.async_copy(msgbuf, acc.at[dst_s], sems, add=True)

    issue_idx(0, 0)
    issue_gath(0, 0)
    issue_idx(1, 1)

    pltpu.sync_copy(zer_hbm, msgbuf)
    for z in range(RPT // ZCH):
        offs = pl.multiple_of(row0 + z * ZCH, 8)
        pltpu.sync_copy(msgbuf, acc.at[pl.ds(offs, ZCH)])
    plsc.subcore_barrier()

    @pl.loop(0, NCHUNK // 2)
    def pair_body(ss):
        s0 = ss * 2

        @pl.when(s0 + 1 < NCHUNK)
        def _():
            issue_gath(s0 + 1, 1)
        finish_chunk(s0, 0)

        @pl.when(s0 + 2 < NCHUNK)
        def _():
            issue_gath(s0 + 2, 0)
        finish_chunk(s0 + 1, 1)

    pltpu.make_async_copy(msgbuf, acc.at[dst_s], sems).wait()
    plsc.subcore_barrier()
    outsem = (semsrc0, semsrc1)
    for z in range(RPT // OCH):
        par = z % 2
        offs = pl.multiple_of(row0 + z * OCH, 8)
        slot = pl.ds(par * OCH, OCH)
        if z >= 2:
            poffs = pl.multiple_of(row0 + (z - 2) * OCH, 8)
            pltpu.make_async_copy(msgbuf.at[slot],
                                  part_hbm.at[cid, pl.ds(poffs, OCH)],
                                  outsem[par]).wait()
        pltpu.sync_copy(acc.at[pl.ds(offs, OCH)], msgbuf.at[slot])
        pltpu.async_copy(msgbuf.at[slot], part_hbm.at[cid, pl.ds(offs, OCH)],
                         outsem[par])
    for z in (RPT // OCH - 2, RPT // OCH - 1):
        par = z % 2
        offs = pl.multiple_of(row0 + z * OCH, 8)
        pltpu.make_async_copy(msgbuf.at[pl.ds(par * OCH, OCH)],
                              part_hbm.at[cid, pl.ds(offs, OCH)],
                              outsem[par]).wait()


def _final_body(part_ref, we_ref, skip_ref, out_ref):
    p = part_ref[0] + part_ref[1]
    num = p[:, :D] + jnp.dot(p[:, D:D + DE], we_ref[...],
                             preferred_element_type=jnp.float32)
    den = p[:, D + DE:D + DE + 1] + jnp.float32(1e-16)
    out_ref[...] = num / den + skip_ref[...]


_final = pl.pallas_call(
    _final_body,
    grid=(N // BN,),
    in_specs=[
        pl.BlockSpec((NC, BN, ROW), lambda i: (0, i, 0)),
        pl.BlockSpec((DE, C), lambda i: (0, 0)),
        pl.BlockSpec((BN, C), lambda i: (i, 0)),
    ],
    out_specs=pl.BlockSpec((BN, C), lambda i: (i, 0)),
    out_shape=jax.ShapeDtypeStruct((N, C), jnp.float32),
)


def kernel(x, edge_index, edge_attr, Wq, bq, Wk, bk, Wv, bv, We, Wskip, bskip):
    perm = jnp.arange(D).reshape(4, 2, 16).transpose(0, 2, 1).reshape(D)
    Wvp = Wv[:, perm]
    bvp = bv[perm]
    WeT_ext = jnp.stack([We, jnp.zeros_like(We)], axis=1).reshape(2 * DE, D).T

    q2, kv, skip = _proj(
        x, Wq, bq.reshape(1, C), Wk, bk.reshape(1, C), Wvp, bvp.reshape(1, C),
        Wskip, bskip.reshape(1, C), WeT_ext)
    zer = jnp.zeros((ZCH, ROW), jnp.float32)
    part = _edge_kernel(q2, kv, edge_attr, edge_index, zer)
    return _final(part, We, skip)


# speedup vs baseline: 1.1457x; 1.1457x over previous
import functools
import math

import jax
import jax.numpy as jnp
from jax import lax
from jax.experimental import pallas as pl
from jax.experimental.pallas import tpu as pltpu
from jax.experimental.pallas import tpu_sc as plsc

N = 10000
E = 320000
D = 128
DE = 16
C = 128

NC, NS, L = 2, 16, 16
NW = NC * NS
EPW = E // NW
CH = 40
NCHUNK = EPW // CH
ROW = 160
NP = 10240
RPT = NP // NS
ZCH = 40
OCH = 16
DKV = 2 * D
DQ2 = D + 2 * DE

BN = 2000


def _proj_body(x_ref, wq, bq, wk, bk, wvp, bvp, wsk, bsk, wet,
               q2_o, kv_o, sk_o):
    xb = x_ref[...]
    q = jnp.dot(xb, wq[...], preferred_element_type=jnp.float32) + bq[...]
    q2_o[:, :D] = q.astype(jnp.bfloat16)
    q2_o[:, D:] = jnp.dot(q, wet[...],
                          preferred_element_type=jnp.float32).astype(jnp.bfloat16)
    kv_o[:, :D] = (jnp.dot(xb, wk[...], preferred_element_type=jnp.float32)
                   + bk[...]).astype(jnp.bfloat16)
    kv_o[:, D:] = (jnp.dot(xb, wvp[...], preferred_element_type=jnp.float32)
                   + bvp[...]).astype(jnp.bfloat16)
    sk_o[...] = jnp.dot(xb, wsk[...], preferred_element_type=jnp.float32) + bsk[...]


_proj = pl.pallas_call(
    _proj_body,
    grid=(N // BN,),
    in_specs=[
        pl.BlockSpec((BN, D), lambda i: (i, 0)),
        pl.BlockSpec((D, C), lambda i: (0, 0)),
        pl.BlockSpec((1, C), lambda i: (0, 0)),
        pl.BlockSpec((D, C), lambda i: (0, 0)),
        pl.BlockSpec((1, C), lambda i: (0, 0)),
        pl.BlockSpec((D, C), lambda i: (0, 0)),
        pl.BlockSpec((1, C), lambda i: (0, 0)),
        pl.BlockSpec((D, C), lambda i: (0, 0)),
        pl.BlockSpec((1, C), lambda i: (0, 0)),
        pl.BlockSpec((D, 2 * DE), lambda i: (0, 0)),
    ],
    out_specs=[
        pl.BlockSpec((BN, DQ2), lambda i: (i, 0)),
        pl.BlockSpec((BN, DKV), lambda i: (i, 0)),
        pl.BlockSpec((BN, C), lambda i: (i, 0)),
    ],
    out_shape=[
        jax.ShapeDtypeStruct((N, DQ2), jnp.bfloat16),
        jax.ShapeDtypeStruct((N, DKV), jnp.bfloat16),
        jax.ShapeDtypeStruct((N, C), jnp.float32),
    ],
)


_mesh = plsc.VectorSubcoreMesh(core_axis_name="c", subcore_axis_name="s",
                               num_cores=NC, num_subcores=NS)

_F = plsc.PackFormat.INTERLEAVED


@functools.partial(
    pl.kernel,
    out_type=jax.ShapeDtypeStruct((NC, NP, ROW), jnp.float32),
    mesh=_mesh,
    compiler_params=pltpu.CompilerParams(needs_layout_passes=False,
                                         use_tc_tiling_on_sc=False),
    scratch_types=[
        pltpu.VMEM((2, CH), jnp.int32),
        pltpu.VMEM((2, CH), jnp.int32),
        pltpu.VMEM((CH,), jnp.int32),
        pltpu.VMEM((2, CH, DQ2), jnp.bfloat16),
        pltpu.VMEM((2, CH, DKV), jnp.bfloat16),
        pltpu.VMEM((2, CH, DE), jnp.float32),
        pltpu.VMEM((CH, ROW), jnp.float32),
        pltpu.VMEM_SHARED((NP, ROW), jnp.float32),
    ] + [pltpu.SemaphoreType.DMA] * 12,
)
def _edge_kernel(q2_hbm, kv_hbm, ea_hbm, ei_hbm,
                 zer_hbm, part_hbm, src_v, dst_v, dst_s, q2buf, kvbuf,
                 eabuf, msgbuf, acc, sems, semd,
                 semsrc0, semsrc1, semdst0, semdst1,
                 semkv0, semkv1, semq20, semq21, semea0, semea1):
    cid = lax.axis_index("c")
    sid = lax.axis_index("s")
    wid = cid * NS + sid
    row0 = sid * RPT

    inv_sqrt_c = jnp.float32(1.0 / math.sqrt(C))
    base_w = wid * EPW
    lane0 = lax.iota(jnp.int32, L) == 0
    lanes = lax.iota(jnp.int32, L)
    shuf = [(lanes ^ o)[:, None] for o in (8, 4, 2, 1)]
    gdn = lax.GatherDimensionNumbers(offset_dims=(), collapsed_slice_dims=(0,),
                                     start_index_map=(0,))

    def lane_perm(x, idx):
        return lax.gather(x, idx, gdn, (1,),
                          mode=lax.GatherScatterMode.PROMISE_IN_BOUNDS)

    semsrc = (semsrc0, semsrc1)
    semdst = (semdst0, semdst1)
    semkv = (semkv0, semkv1)
    semq2 = (semq20, semq21)
    semea = (semea0, semea1)


    def issue_idx(s, par):
        b0 = pl.multiple_of(base_w + s * CH, 8)
        pltpu.async_copy(ei_hbm.at[0, pl.ds(b0, CH)], src_v.at[par],
                         semsrc[par])
        pltpu.async_copy(ei_hbm.at[1, pl.ds(b0, CH)], dst_v.at[par],
                         semdst[par])

    def issue_gath(s, par):
        b0 = pl.multiple_of(base_w + s * CH, 8)
        pltpu.async_copy(ea_hbm.at[pl.ds(b0, CH)], eabuf.at[par], semea[par])
        pltpu.make_async_copy(ei_hbm.at[0, pl.ds(b0, CH)], src_v.at[par],
                              semsrc[par]).wait()
        pltpu.make_async_copy(ei_hbm.at[1, pl.ds(b0, CH)], dst_v.at[par],
                              semdst[par]).wait()
        pltpu.async_copy(kv_hbm.at[src_v.at[par]], kvbuf.at[par], semkv[par])
        pltpu.async_copy(q2_hbm.at[dst_v.at[par]], q2buf.at[par], semq2[par])

    def finish_chunk(s, par):
        pltpu.make_async_copy(kv_hbm.at[src_v.at[par]], kvbuf.at[par],
                              semkv[par]).wait()
        pltpu.make_async_copy(q2_hbm.at[dst_v.at[par]], q2buf.at[par],
                              semq2[par]).wait()
        pltpu.make_async_copy(ea_hbm.at[pl.ds(0, CH)], eabuf.at[par],
                              semea[par]).wait()
        @pl.when(s + 2 < NCHUNK)
        def _():
            issue_idx(s + 2, par)
        @pl.when(s > 0)
        def _():
            pltpu.make_async_copy(msgbuf, acc.at[dst_s], sems).wait()
        b0 = pl.multiple_of(base_w + s * CH, 8)
        cps = pltpu.async_copy(ei_hbm.at[1, pl.ds(b0, CH)], dst_s, semd)

        q2c = q2buf.at[par]
        kvc = kvbuf.at[par]
        eac = eabuf.at[par]

        @plsc.parallel_loop(0, CH, unroll=8)
        def edge_body(e):
            ea_e = eac[e, :]
            qe_a, _ = plsc.unpack(q2c[e, pl.ds(D, 2 * L)], format=_F,
                                  preferred_element_type=jnp.float32)
            part = qe_a * ea_e
            for cc in range(D // (2 * L)):
                qa, qb = plsc.unpack(q2c[e, pl.ds(2 * L * cc, 2 * L)],
                                     format=_F,
                                     preferred_element_type=jnp.float32)
                ka, kb = plsc.unpack(kvc[e, pl.ds(2 * L * cc, 2 * L)],
                                     format=_F,
                                     preferred_element_type=jnp.float32)
                part = part + qa * ka + qb * kb
            for sh in shuf:
                part = part + lane_perm(part, sh)
            ex = jnp.exp(part * inv_sqrt_c)
            for cc in range(D // (2 * L)):
                va, vb = plsc.unpack(kvc[e, pl.ds(D + 2 * L * cc, 2 * L)],
                                     format=_F,
                                     preferred_element_type=jnp.float32)
                msgbuf[e, pl.ds(2 * L * cc, L)] = va * ex
                msgbuf[e, pl.ds(2 * L * cc + L, L)] = vb * ex
            msgbuf[e, pl.ds(D, L)] = ea_e * ex
            msgbuf[e, pl.ds(D + DE, L)] = jnp.where(
                lane0, ex, jnp.zeros((L,), jnp.float32))

        cps.wait()
        pltpu.async_copy(msgbuf, acc.at[dst_s], sems, add=True)

    issue_idx(0, 0)
    issue_gath(0, 0)
    issue_idx(1, 1)

    pltpu.sync_copy(zer_hbm, msgbuf)
    for z in range(RPT // ZCH):
        offs = pl.multiple_of(row0 + z * ZCH, 8)
        pltpu.sync_copy(msgbuf, acc.at[pl.ds(offs, ZCH)])
    plsc.subcore_barrier()

    @pl.loop(0, NCHUNK // 2)
    def pair_body(ss):
        s0 = ss * 2

        @pl.when(s0 + 1 < NCHUNK)
        def _():
            issue_gath(s0 + 1, 1)
        finish_chunk(s0, 0)

        @pl.when(s0 + 2 < NCHUNK)
        def _():
            issue_gath(s0 + 2, 0)
        finish_chunk(s0 + 1, 1)

    pltpu.make_async_copy(msgbuf, acc.at[dst_s], sems).wait()
    plsc.subcore_barrier()
    outsem = (semsrc0, semsrc1)
    for z in range(RPT // OCH):
        par = z % 2
        offs = pl.multiple_of(row0 + z * OCH, 8)
        slot = pl.ds(par * OCH, OCH)
        if z >= 2:
            poffs = pl.multiple_of(row0 + (z - 2) * OCH, 8)
            pltpu.make_async_copy(msgbuf.at[slot],
                                  part_hbm.at[cid, pl.ds(poffs, OCH)],
                                  outsem[par]).wait()
        pltpu.sync_copy(acc.at[pl.ds(offs, OCH)], msgbuf.at[slot])
        pltpu.async_copy(msgbuf.at[slot], part_hbm.at[cid, pl.ds(offs, OCH)],
                         outsem[par])
    for z in (RPT // OCH - 2, RPT // OCH - 1):
        par = z % 2
        offs = pl.multiple_of(row0 + z * OCH, 8)
        pltpu.make_async_copy(msgbuf.at[pl.ds(par * OCH, OCH)],
                              part_hbm.at[cid, pl.ds(offs, OCH)],
                              outsem[par]).wait()


def _final_body(part_ref, we_ref, skip_ref, out_ref):
    p = part_ref[0] + part_ref[1]
    num = p[:, :D] + jnp.dot(p[:, D:D + DE], we_ref[...],
                             preferred_element_type=jnp.float32)
    den = p[:, D + DE:D + DE + 1] + jnp.float32(1e-16)
    out_ref[...] = num / den + skip_ref[...]


_final = pl.pallas_call(
    _final_body,
    grid=(N // BN,),
    in_specs=[
        pl.BlockSpec((NC, BN, ROW), lambda i: (0, i, 0)),
        pl.BlockSpec((DE, C), lambda i: (0, 0)),
        pl.BlockSpec((BN, C), lambda i: (i, 0)),
    ],
    out_specs=pl.BlockSpec((BN, C), lambda i: (i, 0)),
    out_shape=jax.ShapeDtypeStruct((N, C), jnp.float32),
)


def kernel(x, edge_index, edge_attr, Wq, bq, Wk, bk, Wv, bv, We, Wskip, bskip):
    perm = jnp.arange(D).reshape(4, 2, 16).transpose(0, 2, 1).reshape(D)
    Wvp = Wv[:, perm]
    bvp = bv[perm]
    WeT_ext = jnp.stack([We, jnp.zeros_like(We)], axis=1).reshape(2 * DE, D).T

    q2, kv, skip = _proj(
        x, Wq, bq.reshape(1, C), Wk, bk.reshape(1, C), Wvp, bvp.reshape(1, C),
        Wskip, bskip.reshape(1, C), WeT_ext)
    zer = jnp.zeros((ZCH, ROW), jnp.float32)
    part = _edge_kernel(q2, kv, edge_attr, edge_index, zer)
    return _final(part, We, skip)


# speedup vs baseline: 1.1481x; 1.0021x over previous
import functools
import math

import jax
import jax.numpy as jnp
from jax import lax
from jax.experimental import pallas as pl
from jax.experimental.pallas import tpu as pltpu
from jax.experimental.pallas import tpu_sc as plsc

N = 10000
E = 320000
D = 128
DE = 16
C = 128

NC, NS, L = 2, 16, 16
NW = NC * NS
EPW = E // NW
CH = 40
NCHUNK = EPW // CH
ROW = 160
NP = 10240
RPT = NP // NS
ZCH = 40
OCH = 16
DKV = 2 * D
DQ2 = D + 2 * DE

BN = 2000


def _proj_body(x_ref, wq, bq, wk, bk, wvp, bvp, wsk, bsk, wet,
               q2_o, kv_o, sk_o):
    xb = x_ref[...]
    q = jnp.dot(xb, wq[...], preferred_element_type=jnp.float32) + bq[...]
    q2_o[:, :D] = q.astype(jnp.bfloat16)
    q2_o[:, D:] = jnp.dot(q, wet[...],
                          preferred_element_type=jnp.float32).astype(jnp.bfloat16)
    kv_o[:, :D] = (jnp.dot(xb, wk[...], preferred_element_type=jnp.float32)
                   + bk[...]).astype(jnp.bfloat16)
    kv_o[:, D:] = (jnp.dot(xb, wvp[...], preferred_element_type=jnp.float32)
                   + bvp[...]).astype(jnp.bfloat16)
    sk_o[...] = jnp.dot(xb, wsk[...], preferred_element_type=jnp.float32) + bsk[...]


_proj = pl.pallas_call(
    _proj_body,
    grid=(N // BN,),
    in_specs=[
        pl.BlockSpec((BN, D), lambda i: (i, 0)),
        pl.BlockSpec((D, C), lambda i: (0, 0)),
        pl.BlockSpec((1, C), lambda i: (0, 0)),
        pl.BlockSpec((D, C), lambda i: (0, 0)),
        pl.BlockSpec((1, C), lambda i: (0, 0)),
        pl.BlockSpec((D, C), lambda i: (0, 0)),
        pl.BlockSpec((1, C), lambda i: (0, 0)),
        pl.BlockSpec((D, C), lambda i: (0, 0)),
        pl.BlockSpec((1, C), lambda i: (0, 0)),
        pl.BlockSpec((D, 2 * DE), lambda i: (0, 0)),
    ],
    out_specs=[
        pl.BlockSpec((BN, DQ2), lambda i: (i, 0)),
        pl.BlockSpec((BN, DKV), lambda i: (i, 0)),
        pl.BlockSpec((BN, C), lambda i: (i, 0)),
    ],
    out_shape=[
        jax.ShapeDtypeStruct((N, DQ2), jnp.bfloat16),
        jax.ShapeDtypeStruct((N, DKV), jnp.bfloat16),
        jax.ShapeDtypeStruct((N, C), jnp.float32),
    ],
)


_mesh = plsc.VectorSubcoreMesh(core_axis_name="c", subcore_axis_name="s",
                               num_cores=NC, num_subcores=NS)

_F = plsc.PackFormat.INTERLEAVED


@functools.partial(
    pl.kernel,
    out_type=jax.ShapeDtypeStruct((NC, NP, ROW), jnp.float32),
    mesh=_mesh,
    compiler_params=pltpu.CompilerParams(needs_layout_passes=False,
                                         use_tc_tiling_on_sc=False),
    scratch_types=[
        pltpu.VMEM((2, CH), jnp.int32),
        pltpu.VMEM((2, CH), jnp.int32),
        pltpu.VMEM((CH,), jnp.int32),
        pltpu.VMEM((2, CH, DQ2), jnp.bfloat16),
        pltpu.VMEM((2, CH, DKV), jnp.bfloat16),
        pltpu.VMEM((2, CH, DE), jnp.float32),
        pltpu.VMEM((CH, ROW), jnp.float32),
        pltpu.VMEM_SHARED((NP, ROW), jnp.float32),
    ] + [pltpu.SemaphoreType.DMA] * 12,
)
def _edge_kernel(q2_hbm, kv_hbm, ea_hbm, ei_hbm,
                 zer_hbm, part_hbm, src_v, dst_v, dst_s, q2buf, kvbuf,
                 eabuf, msgbuf, acc, sems, semd,
                 semsrc0, semsrc1, semdst0, semdst1,
                 semkv0, semkv1, semq20, semq21, semea0, semea1):
    cid = lax.axis_index("c")
    sid = lax.axis_index("s")
    wid = cid * NS + sid
    row0 = sid * RPT

    inv_sqrt_c = jnp.float32(1.0 / math.sqrt(C))
    base_w = wid * EPW
    lanes = lax.iota(jnp.int32, L)
    shuf = [(lanes ^ o)[:, None] for o in (8, 4, 2, 1)]
    gdn = lax.GatherDimensionNumbers(offset_dims=(), collapsed_slice_dims=(0,),
                                     start_index_map=(0,))

    def lane_perm(x, idx):
        return lax.gather(x, idx, gdn, (1,),
                          mode=lax.GatherScatterMode.PROMISE_IN_BOUNDS)

    semsrc = (semsrc0, semsrc1)
    semdst = (semdst0, semdst1)
    semkv = (semkv0, semkv1)
    semq2 = (semq20, semq21)
    semea = (semea0, semea1)


    def issue_idx(s, par):
        b0 = pl.multiple_of(base_w + s * CH, 8)
        pltpu.async_copy(ei_hbm.at[0, pl.ds(b0, CH)], src_v.at[par],
                         semsrc[par])
        pltpu.async_copy(ei_hbm.at[1, pl.ds(b0, CH)], dst_v.at[par],
                         semdst[par])

    def issue_gath(s, par):
        b0 = pl.multiple_of(base_w + s * CH, 8)
        pltpu.async_copy(ea_hbm.at[pl.ds(b0, CH)], eabuf.at[par], semea[par])
        pltpu.make_async_copy(ei_hbm.at[0, pl.ds(b0, CH)], src_v.at[par],
                              semsrc[par]).wait()
        pltpu.make_async_copy(ei_hbm.at[1, pl.ds(b0, CH)], dst_v.at[par],
                              semdst[par]).wait()
        pltpu.async_copy(kv_hbm.at[src_v.at[par]], kvbuf.at[par], semkv[par])
        pltpu.async_copy(q2_hbm.at[dst_v.at[par]], q2buf.at[par], semq2[par])

    def finish_chunk(s, par):
        pltpu.make_async_copy(kv_hbm.at[src_v.at[par]], kvbuf.at[par],
                              semkv[par]).wait()
        pltpu.make_async_copy(q2_hbm.at[dst_v.at[par]], q2buf.at[par],
                              semq2[par]).wait()
        pltpu.make_async_copy(ea_hbm.at[pl.ds(0, CH)], eabuf.at[par],
                              semea[par]).wait()
        @pl.when(s + 2 < NCHUNK)
        def _():
            issue_idx(s + 2, par)
        @pl.when(s > 0)
        def _():
            pltpu.make_async_copy(msgbuf, acc.at[dst_s], sems).wait()
        b0 = pl.multiple_of(base_w + s * CH, 8)
        cps = pltpu.async_copy(ei_hbm.at[1, pl.ds(b0, CH)], dst_s, semd)

        q2c = q2buf.at[par]
        kvc = kvbuf.at[par]
        eac = eabuf.at[par]

        @plsc.parallel_loop(0, CH, unroll=8)
        def edge_body(e):
            ea_e = eac[e, :]
            qe_a, _ = plsc.unpack(q2c[e, pl.ds(D, 2 * L)], format=_F,
                                  preferred_element_type=jnp.float32)
            part = qe_a * ea_e
            for cc in range(D // (2 * L)):
                qa, qb = plsc.unpack(q2c[e, pl.ds(2 * L * cc, 2 * L)],
                                     format=_F,
                                     preferred_element_type=jnp.float32)
                ka, kb = plsc.unpack(kvc[e, pl.ds(2 * L * cc, 2 * L)],
                                     format=_F,
                                     preferred_element_type=jnp.float32)
                part = part + qa * ka + qb * kb
            for sh in shuf:
                part = part + lane_perm(part, sh)
            ex = jnp.exp(part * inv_sqrt_c)
            for cc in range(D // (2 * L)):
                va, vb = plsc.unpack(kvc[e, pl.ds(D + 2 * L * cc, 2 * L)],
                                     format=_F,
                                     preferred_element_type=jnp.float32)
                msgbuf[e, pl.ds(2 * L * cc, L)] = va * ex
                msgbuf[e, pl.ds(2 * L * cc + L, L)] = vb * ex
            msgbuf[e, pl.ds(D, L)] = ea_e * ex
            msgbuf[e, pl.ds(D + DE, L)] = ex

        cps.wait()
        pltpu.async_copy(msgbuf, acc.at[dst_s], sems, add=True)

    issue_idx(0, 0)
    issue_gath(0, 0)
    issue_idx(1, 1)

    pltpu.sync_copy(zer_hbm, msgbuf)
    for z in range(RPT // ZCH):
        offs = pl.multiple_of(row0 + z * ZCH, 8)
        pltpu.sync_copy(msgbuf, acc.at[pl.ds(offs, ZCH)])
    plsc.subcore_barrier()

    @pl.loop(0, NCHUNK // 2)
    def pair_body(ss):
        s0 = ss * 2

        @pl.when(s0 + 1 < NCHUNK)
        def _():
            issue_gath(s0 + 1, 1)
        finish_chunk(s0, 0)

        @pl.when(s0 + 2 < NCHUNK)
        def _():
            issue_gath(s0 + 2, 0)
        finish_chunk(s0 + 1, 1)

    pltpu.make_async_copy(msgbuf, acc.at[dst_s], sems).wait()
    plsc.subcore_barrier()
    outsem = (semsrc0, semsrc1)
    for z in range(RPT // OCH):
        par = z % 2
        offs = pl.multiple_of(row0 + z * OCH, 8)
        slot = pl.ds(par * OCH, OCH)
        if z >= 2:
            poffs = pl.multiple_of(row0 + (z - 2) * OCH, 8)
            pltpu.make_async_copy(msgbuf.at[slot],
                                  part_hbm.at[cid, pl.ds(poffs, OCH)],
                                  outsem[par]).wait()
        pltpu.sync_copy(acc.at[pl.ds(offs, OCH)], msgbuf.at[slot])
        pltpu.async_copy(msgbuf.at[slot], part_hbm.at[cid, pl.ds(offs, OCH)],
                         outsem[par])
    for z in (RPT // OCH - 2, RPT // OCH - 1):
        par = z % 2
        offs = pl.multiple_of(row0 + z * OCH, 8)
        pltpu.make_async_copy(msgbuf.at[pl.ds(par * OCH, OCH)],
                              part_hbm.at[cid, pl.ds(offs, OCH)],
                              outsem[par]).wait()


def _final_body(part_ref, we_ref, skip_ref, out_ref):
    p = part_ref[0] + part_ref[1]
    num = p[:, :D] + jnp.dot(p[:, D:D + DE], we_ref[...],
                             preferred_element_type=jnp.float32)
    den = p[:, D + DE:D + DE + 1] + jnp.float32(1e-16)
    out_ref[...] = num / den + skip_ref[...]


_final = pl.pallas_call(
    _final_body,
    grid=(N // BN,),
    in_specs=[
        pl.BlockSpec((NC, BN, ROW), lambda i: (0, i, 0)),
        pl.BlockSpec((DE, C), lambda i: (0, 0)),
        pl.BlockSpec((BN, C), lambda i: (i, 0)),
    ],
    out_specs=pl.BlockSpec((BN, C), lambda i: (i, 0)),
    out_shape=jax.ShapeDtypeStruct((N, C), jnp.float32),
)


def kernel(x, edge_index, edge_attr, Wq, bq, Wk, bk, Wv, bv, We, Wskip, bskip):
    perm = jnp.arange(D).reshape(4, 2, 16).transpose(0, 2, 1).reshape(D)
    Wvp = Wv[:, perm]
    bvp = bv[perm]
    WeT_ext = jnp.stack([We, jnp.zeros_like(We)], axis=1).reshape(2 * DE, D).T

    q2, kv, skip = _proj(
        x, Wq, bq.reshape(1, C), Wk, bk.reshape(1, C), Wvp, bvp.reshape(1, C),
        Wskip, bskip.reshape(1, C), WeT_ext)
    zer = jnp.zeros((ZCH, ROW), jnp.float32)
    part = _edge_kernel(q2, kv, edge_attr, edge_index, zer)
    return _final(part, We, skip)
